# baseline (speedup 1.0000x reference)
import functools

import jax
import jax.numpy as jnp
from jax import lax
from jax.experimental import pallas as pl
from jax.experimental.pallas import tpu as pltpu
from jax.experimental.pallas import tpu_sc as plsc

EMB = 64
BATCH = 16384
NROWS = 1000000
NMAIN = (NROWS // 128) * 128
NTAIL = NROWS - NMAIN
NBANDS = 8192
NC = 2
NS = 16
NW = NC * NS
CHUNK = 128
B_PER_W = BATCH // NW
N_CHUNKS = B_PER_W // CHUNK
L = 16

BB = 1302
BK = BB * 128


EMBW = EMB // 2


def _detile_body(t_ref, o_ref):
    x = t_ref[...].reshape(8, 2, BK)
    packed = pltpu.pack_elementwise([x[:, 0, :], x[:, 1, :]],
                                    packed_dtype=jnp.bfloat16)
    y = pltpu.bitcast(packed, jnp.float32)
    o_ref[...] = jnp.swapaxes(y.reshape(8, BB, 128), 0, 1)[None]


@jax.jit
def _detile(tT):
    return pl.pallas_call(
        _detile_body,
        grid=(EMB // 16, NMAIN // BK),
        in_specs=[pl.BlockSpec((16, BK), lambda i, m: (i, m))],
        out_specs=pl.BlockSpec((1, BB, 8, 128), lambda i, m: (i, m, 0, 0)),
        out_shape=jax.ShapeDtypeStruct((EMB // 16, NBANDS, 8, 128),
                                       jnp.float32),
    )(tT)


def _gather_body(idx_hbm, flat_hbm, out_hbm, idx_v, sx_v, cb_v, rows_v, sem):
    wid = lax.axis_index("s") * NC + lax.axis_index("c")
    base = wid * B_PER_W
    pltpu.sync_copy(idx_hbm.at[wid], idx_v)

    for k in range(N_CHUNKS):
        for g in range(CHUNK // L):
            r = idx_v[k, pl.ds(g * L, L)]
            sx_v[k, pl.ds(g * L, L)] = ((r >> 7) << 10) + (r & 127)

    def col_body(c, carry):
        buf = c & 1
        a_c = ((c >> 3) << 23) + ((c & 7) << 7)

        @pl.when(c >= 2)
        def _():
            pltpu.make_async_copy(out_hbm.at[0, pl.ds(0, B_PER_W)],
                                  rows_v.at[c - 2], sem).wait()

        for k in range(N_CHUNKS):
            for g in range(CHUNK // L):
                cb_v[buf, pl.ds(k * CHUNK + g * L, L)] = (
                    sx_v[k, pl.ds(g * L, L)] + a_c)
        pltpu.async_copy(flat_hbm.at[cb_v.at[buf]], rows_v.at[c], sem)

        return carry

    lax.fori_loop(0, EMBW, col_body, 0)

    for c in (EMBW - 2, EMBW - 1):
        pltpu.make_async_copy(out_hbm.at[0, pl.ds(0, B_PER_W)],
                              rows_v.at[c], sem).wait()

    pltpu.sync_copy(rows_v, out_hbm.at[:, pl.ds(base, B_PER_W)])


@jax.jit
def _sc_gather(idx_clamped, flat):
    mesh = plsc.VectorSubcoreMesh(core_axis_name="c", subcore_axis_name="s")
    fn = pl.kernel(
        _gather_body,
        mesh=mesh,
        out_type=jax.ShapeDtypeStruct((EMBW, BATCH), jnp.float32),
        scratch_types=[
            pltpu.VMEM((N_CHUNKS, CHUNK), jnp.int32),
            pltpu.VMEM((N_CHUNKS, CHUNK), jnp.int32),
            pltpu.VMEM((2, B_PER_W), jnp.int32),
            pltpu.VMEM((EMBW, B_PER_W), jnp.float32),
            pltpu.SemaphoreType.DMA,
        ],
        compiler_params=pltpu.CompilerParams(use_tc_tiling_on_sc=False),
    )
    return fn(idx_clamped.reshape(NW, N_CHUNKS, CHUNK), flat)


def _linear_body(e_ref, ix_ref, t_ref, w_ref, b_ref, o_ref):
    bs = e_ref.shape[1]
    wbits = pltpu.bitcast(e_ref[...], jnp.uint32)
    lo = pltpu.bitcast(wbits << 16, jnp.float32)
    hi = pltpu.bitcast(wbits & jnp.uint32(0xFFFF0000), jnp.float32)
    e = jnp.stack([lo, hi], axis=1).reshape(EMB, bs)
    row_iota = lax.broadcasted_iota(jnp.int32, (EMB, bs), 0)
    t = ix_ref[...] - NMAIN
    is_tail = t >= 0
    oneh = jnp.where((row_iota == t) & is_tail, 1.0, 0.0)
    patched = jnp.dot(t_ref[...], oneh, preferred_element_type=jnp.float32)
    e = jnp.where(jnp.broadcast_to(is_tail, (EMB, bs)), patched, e)
    o_ref[...] = jnp.dot(w_ref[...], e,
                         preferred_element_type=jnp.float32) + b_ref[...]


@jax.jit
def _tc_linear(embT, ix, tail, W, b):
    bs = 2048
    return pl.pallas_call(
        _linear_body,
        grid=(BATCH // bs,),
        in_specs=[
            pl.BlockSpec((EMBW, bs), lambda g: (0, g)),
            pl.BlockSpec((1, bs), lambda g: (0, g)),
            pl.BlockSpec((EMB, NTAIL), lambda g: (0, 0)),
            pl.BlockSpec((EMB, EMB), lambda g: (0, 0)),
            pl.BlockSpec((EMB, 1), lambda g: (0, 0)),
        ],
        out_specs=pl.BlockSpec((EMB, bs), lambda g: (0, g)),
        out_shape=jax.ShapeDtypeStruct((EMB, BATCH), jnp.float32),
    )(embT, ix.reshape(1, BATCH), tail, W, b.reshape(EMB, 1))


def kernel(user_indices, item_indices, user_table, item_table, W, b):
    utT = user_table.T
    itT = item_table.T
    uflat = _detile(utT).reshape(-1)
    u_embT = _sc_gather(jnp.minimum(user_indices, NMAIN - 1), uflat)
    iflat = _detile(itT).reshape(-1)
    i_embT = _sc_gather(jnp.minimum(item_indices, NMAIN - 1), iflat)
    u_outT = _tc_linear(u_embT, user_indices, utT[:, NMAIN:], W, b)
    i_outT = _tc_linear(i_embT, item_indices, itT[:, NMAIN:], W, b)
    return (u_outT.T, i_outT.T)


# speedup vs baseline: 1.0043x; 1.0043x over previous
import functools

import jax
import jax.numpy as jnp
from jax import lax
from jax.experimental import pallas as pl
from jax.experimental.pallas import tpu as pltpu
from jax.experimental.pallas import tpu_sc as plsc

EMB = 64
BATCH = 16384
NROWS = 1000000
NMAIN = (NROWS // 128) * 128
NTAIL = NROWS - NMAIN
NBANDS = 8192
NC = 2
NS = 16
NW = NC * NS
CHUNK = 128
B_PER_W = BATCH // NW
N_CHUNKS = B_PER_W // CHUNK
L = 16

BB = 1302
BK = BB * 128


EMBW = EMB // 2


def _detile_body(t_ref, o_ref):
    x = t_ref[...].reshape(8, 2, BK)
    packed = pltpu.pack_elementwise([x[:, 0, :], x[:, 1, :]],
                                    packed_dtype=jnp.bfloat16)
    y = pltpu.bitcast(packed, jnp.float32)
    o_ref[...] = jnp.swapaxes(y.reshape(8, BB, 128), 0, 1)[None]


@jax.jit
def _detile(tT):
    return pl.pallas_call(
        _detile_body,
        grid=(EMB // 16, NMAIN // BK),
        in_specs=[pl.BlockSpec((16, BK), lambda i, m: (i, m))],
        out_specs=pl.BlockSpec((1, BB, 8, 128), lambda i, m: (i, m, 0, 0)),
        out_shape=jax.ShapeDtypeStruct((EMB // 16, NBANDS, 8, 128),
                                       jnp.float32),
    )(tT)


def _gather_body(idx_hbm, flat_hbm, out_hbm, idx_v, sx_v, cb_v, rows_v, sem):
    wid = lax.axis_index("s") * NC + lax.axis_index("c")
    base = wid * B_PER_W
    pltpu.sync_copy(idx_hbm.at[wid], idx_v)

    for k in range(N_CHUNKS):
        for g in range(CHUNK // L):
            r = idx_v[k, pl.ds(g * L, L)]
            sx_v[k, pl.ds(g * L, L)] = ((r >> 7) << 10) + (r & 127)

    def col_body(c, carry):
        buf = c & 1
        a_c = ((c >> 3) << 23) + ((c & 7) << 7)

        @pl.when(c >= 2)
        def _():
            pltpu.make_async_copy(out_hbm.at[0, pl.ds(0, B_PER_W)],
                                  rows_v.at[c - 2], sem).wait()

        for k in range(N_CHUNKS):
            for g in range(CHUNK // L):
                cb_v[buf, k, pl.ds(g * L, L)] = sx_v[k, pl.ds(g * L, L)] + a_c
        for k in range(N_CHUNKS):
            pltpu.async_copy(flat_hbm.at[cb_v.at[buf, k]],
                             rows_v.at[c, pl.ds(k * CHUNK, CHUNK)], sem)

        return carry

    lax.fori_loop(0, EMBW, col_body, 0)

    for c in (EMBW - 2, EMBW - 1):
        pltpu.make_async_copy(out_hbm.at[0, pl.ds(0, B_PER_W)],
                              rows_v.at[c], sem).wait()

    pltpu.sync_copy(rows_v, out_hbm.at[:, pl.ds(base, B_PER_W)])


@jax.jit
def _sc_gather(idx_clamped, flat):
    mesh = plsc.VectorSubcoreMesh(core_axis_name="c", subcore_axis_name="s")
    fn = pl.kernel(
        _gather_body,
        mesh=mesh,
        out_type=jax.ShapeDtypeStruct((EMBW, BATCH), jnp.float32),
        scratch_types=[
            pltpu.VMEM((N_CHUNKS, CHUNK), jnp.int32),
            pltpu.VMEM((N_CHUNKS, CHUNK), jnp.int32),
            pltpu.VMEM((2, N_CHUNKS, CHUNK), jnp.int32),
            pltpu.VMEM((EMBW, B_PER_W), jnp.float32),
            pltpu.SemaphoreType.DMA,
        ],
        compiler_params=pltpu.CompilerParams(use_tc_tiling_on_sc=False),
    )
    return fn(idx_clamped.reshape(NW, N_CHUNKS, CHUNK), flat)


def _linear_body(e_ref, ix_ref, t_ref, w_ref, b_ref, o_ref):
    bs = e_ref.shape[1]
    wbits = pltpu.bitcast(e_ref[...], jnp.uint32)
    lo = pltpu.bitcast(wbits << 16, jnp.float32)
    hi = pltpu.bitcast(wbits & jnp.uint32(0xFFFF0000), jnp.float32)
    e = jnp.stack([lo, hi], axis=1).reshape(EMB, bs)
    row_iota = lax.broadcasted_iota(jnp.int32, (EMB, bs), 0)
    t = ix_ref[...] - NMAIN
    is_tail = t >= 0
    oneh = jnp.where((row_iota == t) & is_tail, 1.0, 0.0)
    patched = jnp.dot(t_ref[...], oneh, preferred_element_type=jnp.float32)
    e = jnp.where(jnp.broadcast_to(is_tail, (EMB, bs)), patched, e)
    o_ref[...] = jnp.dot(w_ref[...], e,
                         preferred_element_type=jnp.float32) + b_ref[...]


@jax.jit
def _tc_linear(embT, ix, tail, W, b):
    bs = 2048
    return pl.pallas_call(
        _linear_body,
        grid=(BATCH // bs,),
        in_specs=[
            pl.BlockSpec((EMBW, bs), lambda g: (0, g)),
            pl.BlockSpec((1, bs), lambda g: (0, g)),
            pl.BlockSpec((EMB, NTAIL), lambda g: (0, 0)),
            pl.BlockSpec((EMB, EMB), lambda g: (0, 0)),
            pl.BlockSpec((EMB, 1), lambda g: (0, 0)),
        ],
        out_specs=pl.BlockSpec((EMB, bs), lambda g: (0, g)),
        out_shape=jax.ShapeDtypeStruct((EMB, BATCH), jnp.float32),
    )(embT, ix.reshape(1, BATCH), tail, W, b.reshape(EMB, 1))


def kernel(user_indices, item_indices, user_table, item_table, W, b):
    utT = user_table.T
    itT = item_table.T
    uflat = _detile(utT).reshape(-1)
    u_embT = _sc_gather(jnp.minimum(user_indices, NMAIN - 1), uflat)
    iflat = _detile(itT).reshape(-1)
    i_embT = _sc_gather(jnp.minimum(item_indices, NMAIN - 1), iflat)
    u_outT = _tc_linear(u_embT, user_indices, utT[:, NMAIN:], W, b)
    i_outT = _tc_linear(i_embT, item_indices, itT[:, NMAIN:], W, b)
    return (u_outT.T, i_outT.T)
